# baseline (speedup 1.0000x reference)
import functools

import jax
import jax.numpy as jnp
from jax import lax
from jax.experimental import pallas as pl
from jax.experimental.pallas import tpu as pltpu
from jax.experimental.pallas import tpu_sc as plsc

B = 16384
D = 64


def _gather_rows_sc(user_ids, item_ids, theta, beta):
    info = plsc.get_sparse_core_info()
    nc, ns = info.num_cores, info.num_subcores
    nw = nc * ns
    bpw = B // nw

    mesh = plsc.VectorSubcoreMesh(core_axis_name="c", subcore_axis_name="s")

    @functools.partial(
        pl.kernel,
        mesh=mesh,
        out_type=(
            jax.ShapeDtypeStruct((B, D), jnp.float32),
            jax.ShapeDtypeStruct((B, D), jnp.float32),
        ),
        scratch_types=[
            pltpu.VMEM((bpw,), jnp.int32),
            pltpu.VMEM((bpw,), jnp.int32),
            pltpu.SemaphoreType.DMA,
            pltpu.SemaphoreType.DMA,
        ],
    )
    def gather_kernel(uid_hbm, iid_hbm, theta_hbm, beta_hbm, out_t, out_b,
                      uidx_v, iidx_v, sem_t, sem_b):
        wid = lax.axis_index("s") * nc + lax.axis_index("c")
        base = wid * bpw
        pltpu.sync_copy(uid_hbm.at[pl.ds(base, bpw)], uidx_v)
        pltpu.sync_copy(iid_hbm.at[pl.ds(base, bpw)], iidx_v)

        def body(g, carry):
            j0 = g * 16
            uvec = uidx_v[pl.ds(j0, 16)]
            ivec = iidx_v[pl.ds(j0, 16)]
            for l in range(16):
                pltpu.async_copy(theta_hbm.at[pl.ds(uvec[l], 1), :],
                                 out_t.at[pl.ds(base + j0 + l, 1), :], sem_t)
                pltpu.async_copy(beta_hbm.at[pl.ds(ivec[l], 1), :],
                                 out_b.at[pl.ds(base + j0 + l, 1), :], sem_b)
            return carry

        lax.fori_loop(0, bpw // 16, body, 0)
        pltpu.make_async_copy(theta_hbm.at[pl.ds(0, bpw), :],
                              out_t.at[pl.ds(base, bpw), :], sem_t).wait()
        pltpu.make_async_copy(beta_hbm.at[pl.ds(0, bpw), :],
                              out_b.at[pl.ds(base, bpw), :], sem_b).wait()

    return gather_kernel(user_ids, item_ids, theta, beta)


def _softplus(x):
    return jnp.maximum(x, 0.0) + jnp.log(1.0 + jnp.exp(-jnp.abs(x)))


def _dot_body(t_ref, b_ref, o_ref):
    sp_t = _softplus(t_ref[...])
    sp_b = _softplus(b_ref[...])
    o_ref[...] = jnp.sum(sp_t * sp_b, axis=1)


def _tc_reduce(trows, brows):
    blk = 2048
    return pl.pallas_call(
        _dot_body,
        grid=(B // blk,),
        in_specs=[
            pl.BlockSpec((blk, D), lambda i: (i, 0)),
            pl.BlockSpec((blk, D), lambda i: (i, 0)),
        ],
        out_specs=pl.BlockSpec((blk,), lambda i: (i,)),
        out_shape=jax.ShapeDtypeStruct((B,), jnp.float32),
    )(trows, brows)


def kernel(user_ids, item_ids, theta_uncons, beta_uncons):
    uid = user_ids.astype(jnp.int32)
    iid = item_ids.astype(jnp.int32)
    trows, brows = _gather_rows_sc(uid, iid, theta_uncons, beta_uncons)
    return _tc_reduce(trows, brows)


# speedup vs baseline: 5.3218x; 5.3218x over previous
import functools

import jax
import jax.numpy as jnp
from jax import lax
from jax.experimental import pallas as pl
from jax.experimental.pallas import tpu as pltpu
from jax.experimental.pallas import tpu_sc as plsc

B = 16384
D = 64
N = 100000
CHUNK = 1024


def _gather_cols_sc(user_ids, item_ids, tt, bt):
    info = plsc.get_sparse_core_info()
    nc, ns = info.num_cores, info.num_subcores
    nw = nc * ns
    rows_per_w = D // nw

    mesh = plsc.VectorSubcoreMesh(core_axis_name="c", subcore_axis_name="s")

    @functools.partial(
        pl.kernel,
        mesh=mesh,
        out_type=(
            jax.ShapeDtypeStruct((D, B), jnp.float32),
            jax.ShapeDtypeStruct((D, B), jnp.float32),
        ),
        scratch_types=[
            pltpu.VMEM((N,), jnp.float32),
            pltpu.VMEM((CHUNK,), jnp.int32),
            pltpu.VMEM((CHUNK,), jnp.float32),
        ],
        compiler_params=pltpu.CompilerParams(needs_layout_passes=False),
    )
    def gather_kernel(uid_hbm, iid_hbm, tt_hbm, bt_hbm, out_t, out_b,
                      rowb, idxb, outb):
        wid = lax.axis_index("s") * nc + lax.axis_index("c")
        zero16 = jnp.zeros((16,), jnp.int32)

        for tab_hbm, idx_hbm, out_hbm in ((tt_hbm, uid_hbm, out_t),
                                          (bt_hbm, iid_hbm, out_b)):
            for kk in range(rows_per_w):
                k = wid * rows_per_w + kk
                pltpu.sync_copy(tab_hbm.at[k], rowb)

                def chunk_body(c, carry):
                    pltpu.sync_copy(idx_hbm.at[pl.ds(c * CHUNK, CHUNK)], idxb)

                    def g_body(g, carry2):
                        j = g * 64
                        for s in range(4):
                            iv = idxb[pl.ds(j + s * 16, 16)]
                            outb[pl.ds(j + s * 16, 16)] = (
                                plsc.load_gather(rowb, [iv]))
                        return carry2

                    lax.fori_loop(0, CHUNK // 64, g_body, 0)
                    pltpu.sync_copy(
                        outb,
                        out_hbm.at[k, pl.ds(c * CHUNK, CHUNK)])
                    return carry

                lax.fori_loop(0, B // CHUNK, chunk_body, 0)

    return gather_kernel(user_ids, item_ids, tt, bt)


def _softplus(x):
    return jnp.maximum(x, 0.0) + jnp.log(1.0 + jnp.exp(-jnp.abs(x)))


def _dot_body(t_ref, b_ref, o_ref):
    sp_t = _softplus(t_ref[...])
    sp_b = _softplus(b_ref[...])
    o_ref[...] = jnp.sum(sp_t * sp_b, axis=0)


def _tc_reduce(trows, brows):
    blk = 2048
    return pl.pallas_call(
        _dot_body,
        grid=(B // blk,),
        in_specs=[
            pl.BlockSpec((D, blk), lambda i: (0, i)),
            pl.BlockSpec((D, blk), lambda i: (0, i)),
        ],
        out_specs=pl.BlockSpec((blk,), lambda i: (i,)),
        out_shape=jax.ShapeDtypeStruct((B,), jnp.float32),
    )(trows, brows)


def kernel(user_ids, item_ids, theta_uncons, beta_uncons):
    uid = user_ids.astype(jnp.int32)
    iid = item_ids.astype(jnp.int32)
    trows, brows = _gather_cols_sc(uid, iid, theta_uncons.T, beta_uncons.T)
    return _tc_reduce(trows, brows)


# speedup vs baseline: 8.2237x; 1.5453x over previous
import functools

import jax
import jax.numpy as jnp
from jax import lax
from jax.experimental import pallas as pl
from jax.experimental.pallas import tpu as pltpu
from jax.experimental.pallas import tpu_sc as plsc

B = 16384
D = 64
N = 100000
CHUNK = 4096
NCHUNK = B // CHUNK


def _gather_cols_sc(user_ids, item_ids, tt, bt):
    info = plsc.get_sparse_core_info()
    nc, ns = info.num_cores, info.num_subcores
    nw = nc * ns
    rows_per_w = D // nw

    mesh = plsc.VectorSubcoreMesh(core_axis_name="c", subcore_axis_name="s")

    @functools.partial(
        pl.kernel,
        mesh=mesh,
        out_type=(
            jax.ShapeDtypeStruct((D, B), jnp.float32),
            jax.ShapeDtypeStruct((D, B), jnp.float32),
        ),
        scratch_types=[
            pltpu.VMEM((N,), jnp.float32),
            pltpu.VMEM((CHUNK,), jnp.int32),
            pltpu.VMEM((CHUNK,), jnp.int32),
            pltpu.VMEM((CHUNK,), jnp.float32),
            pltpu.VMEM((CHUNK,), jnp.float32),
            pltpu.SemaphoreType.DMA,
            pltpu.SemaphoreType.DMA,
            pltpu.SemaphoreType.DMA,
            pltpu.SemaphoreType.DMA,
        ],
        compiler_params=pltpu.CompilerParams(needs_layout_passes=False),
    )
    def gather_kernel(uid_hbm, iid_hbm, tt_hbm, bt_hbm, out_t, out_b,
                      rowb, idxb0, idxb1, outb0, outb1,
                      sem_i0, sem_i1, sem_w0, sem_w1):
        wid = lax.axis_index("s") * nc + lax.axis_index("c")
        idxb = (idxb0, idxb1)
        outb = (outb0, outb1)
        sem_i = (sem_i0, sem_i1)
        sem_w = (sem_w0, sem_w1)
        pend_w = [None, None]

        for tab_hbm, idx_hbm, out_hbm in ((tt_hbm, uid_hbm, out_t),
                                          (bt_hbm, iid_hbm, out_b)):
            for kk in range(rows_per_w):
                k = wid * rows_per_w + kk
                pltpu.sync_copy(tab_hbm.at[k], rowb)
                pend_i = [None, None]
                pend_i[0] = pltpu.async_copy(
                    idx_hbm.at[pl.ds(0, CHUNK)], idxb[0], sem_i[0])
                for c in range(NCHUNK):
                    par = c % 2
                    pend_i[par].wait()
                    if c + 1 < NCHUNK:
                        pend_i[1 - par] = pltpu.async_copy(
                            idx_hbm.at[pl.ds((c + 1) * CHUNK, CHUNK)],
                            idxb[1 - par], sem_i[1 - par])
                    if pend_w[par] is not None:
                        pend_w[par].wait()
                        pend_w[par] = None

                    src_i = idxb[par]
                    dst_o = outb[par]

                    def g_body(g, carry, src_i=src_i, dst_o=dst_o):
                        j = g * 128
                        for s in range(8):
                            iv = src_i[pl.ds(j + s * 16, 16)]
                            dst_o[pl.ds(j + s * 16, 16)] = (
                                plsc.load_gather(rowb, [iv]))
                        return carry

                    lax.fori_loop(0, CHUNK // 128, g_body, 0)
                    pend_w[par] = pltpu.async_copy(
                        dst_o, out_hbm.at[k, pl.ds(c * CHUNK, CHUNK)],
                        sem_w[par])
        for par in range(2):
            if pend_w[par] is not None:
                pend_w[par].wait()

    return gather_kernel(user_ids, item_ids, tt, bt)


def _softplus(x):
    return jnp.maximum(x, 0.0) + jnp.log(1.0 + jnp.exp(-jnp.abs(x)))


def _dot_body(t_ref, b_ref, o_ref):
    sp_t = _softplus(t_ref[...])
    sp_b = _softplus(b_ref[...])
    o_ref[...] = jnp.sum(sp_t * sp_b, axis=0)


def _tc_reduce(trows, brows):
    blk = 2048
    return pl.pallas_call(
        _dot_body,
        grid=(B // blk,),
        in_specs=[
            pl.BlockSpec((D, blk), lambda i: (0, i)),
            pl.BlockSpec((D, blk), lambda i: (0, i)),
        ],
        out_specs=pl.BlockSpec((blk,), lambda i: (i,)),
        out_shape=jax.ShapeDtypeStruct((B,), jnp.float32),
    )(trows, brows)


def kernel(user_ids, item_ids, theta_uncons, beta_uncons):
    uid = user_ids.astype(jnp.int32)
    iid = item_ids.astype(jnp.int32)
    trows, brows = _gather_cols_sc(uid, iid, theta_uncons.T, beta_uncons.T)
    return _tc_reduce(trows, brows)


# speedup vs baseline: 8.7631x; 1.0656x over previous
import functools

import jax
import jax.numpy as jnp
from jax import lax
from jax.experimental import pallas as pl
from jax.experimental.pallas import tpu as pltpu
from jax.experimental.pallas import tpu_sc as plsc

B = 16384
D = 64
N = 100000
CHUNK = 4096
NCHUNK = B // CHUNK


def _gather_cols_sc(user_ids, item_ids, tt, bt):
    info = plsc.get_sparse_core_info()
    nc, ns = info.num_cores, info.num_subcores
    nw = nc * ns
    rows_per_w = D // nw

    mesh = plsc.VectorSubcoreMesh(core_axis_name="c", subcore_axis_name="s")

    @functools.partial(
        pl.kernel,
        mesh=mesh,
        out_type=(
            jax.ShapeDtypeStruct((D, B), jnp.float32),
            jax.ShapeDtypeStruct((D, B), jnp.float32),
        ),
        scratch_types=[
            pltpu.VMEM((N,), jnp.float32),
            pltpu.VMEM((CHUNK,), jnp.int32),
            pltpu.VMEM((CHUNK,), jnp.int32),
            pltpu.VMEM((CHUNK,), jnp.float32),
            pltpu.VMEM((CHUNK,), jnp.float32),
            pltpu.SemaphoreType.DMA,
            pltpu.SemaphoreType.DMA,
            pltpu.SemaphoreType.DMA,
            pltpu.SemaphoreType.DMA,
        ],
        compiler_params=pltpu.CompilerParams(needs_layout_passes=False),
    )
    def gather_kernel(uid_hbm, iid_hbm, tt_hbm, bt_hbm, out_t, out_b,
                      rowb, idxb0, idxb1, outb0, outb1,
                      sem_i0, sem_i1, sem_w0, sem_w1):
        wid = lax.axis_index("s") * nc + lax.axis_index("c")
        idxb = (idxb0, idxb1)
        outb = (outb0, outb1)
        sem_i = (sem_i0, sem_i1)
        sem_w = (sem_w0, sem_w1)
        pend_w = [None, None]

        for tab_hbm, idx_hbm, out_hbm in ((tt_hbm, uid_hbm, out_t),
                                          (bt_hbm, iid_hbm, out_b)):
            for kk in range(rows_per_w):
                k = wid * rows_per_w + kk
                pltpu.sync_copy(tab_hbm.at[k], rowb)
                pend_i = [None, None]
                pend_i[0] = pltpu.async_copy(
                    idx_hbm.at[pl.ds(0, CHUNK)], idxb[0], sem_i[0])
                for c in range(NCHUNK):
                    par = c % 2
                    pend_i[par].wait()
                    if c + 1 < NCHUNK:
                        pend_i[1 - par] = pltpu.async_copy(
                            idx_hbm.at[pl.ds((c + 1) * CHUNK, CHUNK)],
                            idxb[1 - par], sem_i[1 - par])
                    if pend_w[par] is not None:
                        pend_w[par].wait()
                        pend_w[par] = None

                    src_i = idxb[par]
                    dst_o = outb[par]

                    @plsc.parallel_loop(0, CHUNK // 16, unroll=8)
                    def g_body(g, src_i=src_i, dst_o=dst_o):
                        j = g * 16
                        iv = src_i[pl.ds(j, 16)]
                        dst_o[pl.ds(j, 16)] = plsc.load_gather(rowb, [iv])
                    pend_w[par] = pltpu.async_copy(
                        dst_o, out_hbm.at[k, pl.ds(c * CHUNK, CHUNK)],
                        sem_w[par])
        for par in range(2):
            if pend_w[par] is not None:
                pend_w[par].wait()

    return gather_kernel(user_ids, item_ids, tt, bt)


def _softplus(x):
    return jnp.maximum(x, 0.0) + jnp.log(1.0 + jnp.exp(-jnp.abs(x)))


def _dot_body(t_ref, b_ref, o_ref):
    sp_t = _softplus(t_ref[...])
    sp_b = _softplus(b_ref[...])
    o_ref[...] = jnp.sum(sp_t * sp_b, axis=0)


def _tc_reduce(trows, brows):
    blk = 2048
    return pl.pallas_call(
        _dot_body,
        grid=(B // blk,),
        in_specs=[
            pl.BlockSpec((D, blk), lambda i: (0, i)),
            pl.BlockSpec((D, blk), lambda i: (0, i)),
        ],
        out_specs=pl.BlockSpec((blk,), lambda i: (i,)),
        out_shape=jax.ShapeDtypeStruct((B,), jnp.float32),
    )(trows, brows)


def kernel(user_ids, item_ids, theta_uncons, beta_uncons):
    uid = user_ids.astype(jnp.int32)
    iid = item_ids.astype(jnp.int32)
    trows, brows = _gather_cols_sc(uid, iid, theta_uncons.T, beta_uncons.T)
    return _tc_reduce(trows, brows)
